# Initial kernel scaffold; baseline (speedup 1.0000x reference)
import jax
import jax.numpy as jnp
from jax.experimental import pallas as pl


def kernel(x, edge_index, edge_attr, batch, gin_params, att_W, att_b, fc1_W, fc1_b, bn_g, bn_b, fc2_W, fc2_b):
    raise NotImplementedError("write your pallas kernel here")


# speedup vs baseline: 3.6138x; 3.6138x over previous
import functools

import jax
import jax.numpy as jnp
from jax import lax
from jax.experimental import pallas as pl
from jax.experimental.pallas import tpu as pltpu
from jax.experimental.pallas import tpu_sc as plsc

_N = 10000
_E = 320000
_D = 128
_H = 256
_G = 64
_NB = 10
_BR = _N // _NB
_K = 80
_NSUB = 16
_RPS = 624
_RTAIL = _N - _RPS * _NSUB
_ZR = 208


def _sc_segment_sum(tab, src, dst, *, feat_split):
    edges_per_sub = _E // _NSUB if feat_split else _E // (2 * _NSUB)
    nchunks = edges_per_sub // _K
    mesh = plsc.VectorSubcoreMesh(core_axis_name="c", subcore_axis_name="s")

    @functools.partial(
        pl.kernel,
        mesh=mesh,
        out_type=jax.ShapeDtypeStruct((2, _N, 128), jnp.float32),
        scratch_types=[
            pltpu.VMEM((_K,), jnp.int32),
            pltpu.VMEM((_K,), jnp.int32),
            pltpu.VMEM((_K, 128), jnp.float32),
            pltpu.VMEM((_ZR, 128), jnp.float32),
            pltpu.VMEM_SHARED((_N, 128), jnp.float32),
            pltpu.SemaphoreType.DMA,
        ],
    )
    def k(tab_hbm, src_hbm, dst_hbm, out_hbm, src_v, dst_v, rows_v, zbuf,
          agg_sh, sem):
        c = lax.axis_index("c")
        s = lax.axis_index("s")

        @pl.loop(0, _ZR)
        def _(i):
            @pl.loop(0, 128, step=16)
            def _(j):
                zbuf[i, pl.ds(j, 16)] = jnp.zeros((16,), jnp.float32)

        @pl.loop(0, _RPS, step=_ZR)
        def _(r):
            pltpu.sync_copy(zbuf, agg_sh.at[pl.ds(s * _RPS + r, _ZR)])

        @pl.when(s == 0)
        def _():
            pltpu.sync_copy(zbuf.at[pl.ds(0, _RTAIL)],
                            agg_sh.at[pl.ds(_RPS * _NSUB, _RTAIL)])

        plsc.subcore_barrier()

        if feat_split:
            base = s * edges_per_sub
        else:
            base = (c * _NSUB + s) * edges_per_sub

        @pl.loop(0, nchunks)
        def _(t):
            e0 = base + t * _K
            pltpu.sync_copy(src_hbm.at[pl.ds(e0, _K)], src_v)
            pltpu.sync_copy(dst_hbm.at[pl.ds(e0, _K)], dst_v)
            if feat_split:
                pltpu.async_copy(tab_hbm.at[c].at[src_v], rows_v, sem).wait()
            else:
                pltpu.async_copy(tab_hbm.at[src_v], rows_v, sem).wait()
            pltpu.sync_copy(rows_v, agg_sh.at[dst_v], add=True)

        plsc.subcore_barrier()
        pltpu.sync_copy(
            agg_sh.at[pl.ds(s * _RPS, _RPS)],
            out_hbm.at[c, pl.ds(s * _RPS, _RPS)],
        )

        @pl.when(s == 0)
        def _():
            pltpu.sync_copy(
                agg_sh.at[pl.ds(_RPS * _NSUB, _RTAIL)],
                out_hbm.at[c, pl.ds(_RPS * _NSUB, _RTAIL)],
            )

    return k(tab, src, dst)


def _mm_t(a, w):
    return lax.dot_general(a.astype(jnp.bfloat16), w.astype(jnp.bfloat16),
                           (((1,), (1,)), ((), ())),
                           preferred_element_type=jnp.float32)


def _tc_gin_layer(h, agg, W1, b1, W2, b2, gamma, beta, *, first):
    din = _D if first else _H

    def body(h_ref, a_ref, w1_ref, b1_ref, w2_ref, b2_ref, g_ref, be_ref,
             out_ref, z_sc, sum_sc, sq_sc):
        p = pl.program_id(0)
        j = pl.program_id(1)

        @pl.when(p == 0)
        def _():
            if first:
                u = h_ref[...] + a_ref[0] + a_ref[1]
            else:
                u = (jnp.concatenate([h_ref[0], h_ref[1]], axis=1)
                     + jnp.concatenate([a_ref[0], a_ref[1]], axis=1))
            z = jnp.maximum(_mm_t(u, w1_ref[...]) + b1_ref[...], 0.0)
            z = jnp.maximum(_mm_t(z, w2_ref[...]) + b2_ref[...], 0.0)
            z_sc[pl.ds(j * _BR, _BR), :] = z
            cs = jnp.sum(z, axis=0, keepdims=True)
            cq = jnp.sum(z * z, axis=0, keepdims=True)

            @pl.when(j == 0)
            def _():
                sum_sc[...] = cs
                sq_sc[...] = cq

            @pl.when(j > 0)
            def _():
                sum_sc[...] = sum_sc[...] + cs
                sq_sc[...] = sq_sc[...] + cq

        @pl.when(p == 1)
        def _():
            m = sum_sc[...] / _N
            v = sq_sc[...] / _N - m * m
            sc = g_ref[...] * lax.rsqrt(v + 1e-5)
            z = z_sc[pl.ds(j * _BR, _BR), :]
            hn = (z - m) * sc + be_ref[...]
            out_ref[0] = hn[:, :128]
            out_ref[1] = hn[:, 128:]

    if first:
        h_spec = pl.BlockSpec((_BR, _D), lambda p, j: (j, 0))
    else:
        h_spec = pl.BlockSpec((2, _BR, 128), lambda p, j: (0, j, 0))

    return pl.pallas_call(
        body,
        grid=(2, _NB),
        in_specs=[
            h_spec,
            pl.BlockSpec((2, _BR, 128), lambda p, j: (0, j, 0)),
            pl.BlockSpec((_H, din), lambda p, j: (0, 0)),
            pl.BlockSpec((1, _H), lambda p, j: (0, 0)),
            pl.BlockSpec((_H, _H), lambda p, j: (0, 0)),
            pl.BlockSpec((1, _H), lambda p, j: (0, 0)),
            pl.BlockSpec((1, _H), lambda p, j: (0, 0)),
            pl.BlockSpec((1, _H), lambda p, j: (0, 0)),
        ],
        out_specs=pl.BlockSpec((2, _BR, 128), lambda p, j: (0, j, 0)),
        out_shape=jax.ShapeDtypeStruct((2, _N, 128), jnp.float32),
        scratch_shapes=[
            pltpu.VMEM((_N, _H), jnp.float32),
            pltpu.VMEM((1, _H), jnp.float32),
            pltpu.VMEM((1, _H), jnp.float32),
        ],
    )(h, agg, W1, b1, W2, b2, gamma, beta)


def _tc_pool(h, batch3, att_W, att_b, fc1_W, fc1_b, bn_g, bn_b, fc2_W, fc2_b):
    def body(h_ref, b_ref, aw_ref, ab_ref, f1w_ref, f1b_ref, bg_ref, bb_ref,
             f2w_ref, f2b_ref, out_ref, p_sc, den_sc):
        j = pl.program_id(0)
        hcat = jnp.concatenate([h_ref[0], h_ref[1]], axis=1)
        s_row = lax.dot_general(aw_ref[...].astype(jnp.bfloat16),
                                hcat.astype(jnp.bfloat16),
                                (((1,), (1,)), ((), ())),
                                preferred_element_type=jnp.float32)
        s_row = s_row + ab_ref[0, 0]
        e_row = jnp.exp(s_row)
        gids = lax.broadcasted_iota(jnp.int32, (_G, _BR), 0)
        mask = b_ref[0] == gids
        w = jnp.where(mask, jnp.broadcast_to(e_row, (_G, _BR)), 0.0)
        p_blk = lax.dot_general(w, hcat, (((1,), (0,)), ((), ())),
                                precision=lax.Precision.HIGHEST,
                                preferred_element_type=jnp.float32)
        d_blk = jnp.sum(w, axis=1, keepdims=True)

        @pl.when(j == 0)
        def _():
            p_sc[...] = p_blk
            den_sc[...] = d_blk

        @pl.when(j > 0)
        def _():
            p_sc[...] = p_sc[...] + p_blk
            den_sc[...] = den_sc[...] + d_blk

        @pl.when(j == _NB - 1)
        def _():
            pooled = p_sc[...] / den_sc[...]
            z1 = jnp.maximum(_mm_t(pooled, f1w_ref[...]) + f1b_ref[...], 0.0)
            m = jnp.mean(z1, axis=0, keepdims=True)
            v = jnp.mean(z1 * z1, axis=0, keepdims=True) - m * m
            zn = bg_ref[...] * (z1 - m) * lax.rsqrt(v + 1e-5) + bb_ref[...]
            out_ref[...] = _mm_t(zn, f2w_ref[...]) + f2b_ref[...]

    return pl.pallas_call(
        body,
        grid=(_NB,),
        in_specs=[
            pl.BlockSpec((2, _BR, 128), lambda j: (0, j, 0)),
            pl.BlockSpec((1, 1, _BR), lambda j: (j, 0, 0)),
            pl.BlockSpec((1, _H), lambda j: (0, 0)),
            pl.BlockSpec((1, 1), lambda j: (0, 0)),
            pl.BlockSpec((_H // 4, _H), lambda j: (0, 0)),
            pl.BlockSpec((1, _H // 4), lambda j: (0, 0)),
            pl.BlockSpec((1, _H // 4), lambda j: (0, 0)),
            pl.BlockSpec((1, _H // 4), lambda j: (0, 0)),
            pl.BlockSpec((2, _H // 4), lambda j: (0, 0)),
            pl.BlockSpec((1, 2), lambda j: (0, 0)),
        ],
        out_specs=pl.BlockSpec((_G, 2), lambda j: (0, 0)),
        out_shape=jax.ShapeDtypeStruct((_G, 2), jnp.float32),
        scratch_shapes=[
            pltpu.VMEM((_G, _H), jnp.float32),
            pltpu.VMEM((_G, 1), jnp.float32),
        ],
    )(h, batch3, att_W, att_b, fc1_W, fc1_b, bn_g, bn_b, fc2_W, fc2_b)


def kernel(x, edge_index, edge_attr, batch, gin_params, att_W, att_b,
           fc1_W, fc1_b, bn_g, bn_b, fc2_W, fc2_b):
    del edge_attr
    src = edge_index[0]
    dst = edge_index[1]
    h = None
    for l, (W1, b1, W2, b2, gamma, beta) in enumerate(gin_params):
        first = l == 0
        tab = x if first else h
        agg = _sc_segment_sum(tab, src, dst, feat_split=not first)
        h = _tc_gin_layer(
            x if first else h, agg,
            W1, b1.reshape(1, -1), W2, b2.reshape(1, -1),
            gamma.reshape(1, -1), beta.reshape(1, -1), first=first)
    batch3 = batch.reshape(_NB, 1, _BR)
    return _tc_pool(h, batch3, att_W, att_b.reshape(1, 1),
                    fc1_W, fc1_b.reshape(1, -1), bn_g.reshape(1, -1),
                    bn_b.reshape(1, -1), fc2_W, fc2_b.reshape(1, -1))
